# baseline (speedup 1.0000x reference)
import functools
import jax
import jax.numpy as jnp
from jax import lax
from jax.experimental import pallas as pl
from jax.experimental.pallas import tpu as pltpu
from jax.experimental.pallas import tpu_sc as plsc

_B = 4096
_L = 200
_D = 64
_GV = 1000
_IV = 1000000
_ROWS = _B * _L
_LANES = 16
_RW = 8192


def _relayout_body(tin_ref, out_ref):
    x = tin_ref[...]
    y = jnp.swapaxes(x, 0, 1)
    y3 = y.reshape(_RW // 2, 2, _D)
    out_ref[...] = jnp.concatenate([y3[:, 0, :], y3[:, 1, :]], axis=1)


def _tc_relayout(item_table_t):
    return pl.pallas_call(
        _relayout_body,
        grid=(pl.cdiv(_IV, _RW),),
        in_specs=[pl.BlockSpec((_D, _RW), lambda i: (0, i))],
        out_specs=pl.BlockSpec((_RW // 2, 128), lambda i: (i, 0)),
        out_shape=jax.ShapeDtypeStruct((_IV // 2, 128), jnp.float32),
    )(item_table_t)


def _prep_body(genre_ref, w_ref, b_ref, pos_ref, pg_ref, pb_ref):
    pg_ref[...] = jnp.dot(genre_ref[...], w_ref[...],
                          preferred_element_type=jnp.float32)
    pb_ref[...] = pos_ref[...] + b_ref[...]


def _tc_prep(genre_table, proj_W, proj_b, pos_table):
    return pl.pallas_call(
        _prep_body,
        out_shape=(
            jax.ShapeDtypeStruct((_GV, _D), jnp.float32),
            jax.ShapeDtypeStruct((_L, _D), jnp.float32),
        ),
    )(genre_table, proj_W, proj_b.reshape(1, _D), pos_table)


def _make_sc_kernel():
    info = plsc.get_sparse_core_info()
    nc, ns = info.num_cores, info.num_subcores
    nw = nc * ns
    rpw = _ROWS // nw
    ch = 128
    nch = rpw // ch

    mesh = plsc.VectorSubcoreMesh(core_axis_name="c", subcore_axis_name="s")

    @functools.partial(
        pl.kernel,
        mesh=mesh,
        out_type=jax.ShapeDtypeStruct((_ROWS, _D), jnp.float32),
        compiler_params=pltpu.CompilerParams(use_tc_tiling_on_sc=False),
        scratch_types=[
            pltpu.VMEM((rpw,), jnp.int32),
            pltpu.VMEM((rpw,), jnp.int32),
            pltpu.VMEM((ch, _D), jnp.float32),
            pltpu.VMEM((ch, _D), jnp.float32),
            pltpu.VMEM((ch, _D), jnp.float32),
            pltpu.VMEM((ch, _D), jnp.float32),
            pltpu.VMEM((2 * _L, _D), jnp.float32),
            pltpu.VMEM_SHARED((_GV, _D), jnp.float32),
            pltpu.SemaphoreType.DMA,
            pltpu.SemaphoreType.DMA,
            pltpu.SemaphoreType.DMA,
            pltpu.SemaphoreType.DMA,
            pltpu.SemaphoreType.DMA,
            pltpu.SemaphoreType.DMA,
        ],
    )
    def sc_kernel(item_tab, pg_tab, pb_tab, item_idx, genre_idx, out,
                  idx_i, idx_g,
                  rows_i0, rows_g0, rows_i1, rows_g1,
                  pos_v, pg_sh, s_gi0, s_gg0, s_gi1, s_gg1, s_sc0, s_sc1):
        cid = lax.axis_index("c")
        sid = lax.axis_index("s")
        w = sid * nc + cid
        base = w * rpw

        @pl.when(sid == 0)
        def _fill_shared():
            pltpu.sync_copy(pg_tab, pg_sh)

        pltpu.sync_copy(pb_tab, pos_v.at[pl.ds(0, _L)])
        pltpu.sync_copy(pb_tab, pos_v.at[pl.ds(_L, _L)])
        pltpu.sync_copy(item_idx.at[pl.ds(base, rpw)], idx_i)
        pltpu.sync_copy(genre_idx.at[pl.ds(base, rpw)], idx_g)
        plsc.subcore_barrier()

        sets = ((rows_i0, rows_g0, s_gi0, s_gg0, s_sc0),
                (rows_i1, rows_g1, s_gi1, s_gg1, s_sc1))

        def fetch(c, st):
            rows_i, rows_g, s_gi, s_gg, _ = st
            o = c * ch
            pltpu.async_copy(item_tab.at[idx_i.at[pl.ds(o, ch)]], rows_i, s_gi)
            pltpu.async_copy(pg_sh.at[idx_g.at[pl.ds(o, ch)]], rows_g, s_gg)

        def process(c, st):
            rows_i, rows_g, s_gi, s_gg, s_sc = st
            o = c * ch
            r0 = base + o
            pltpu.make_async_copy(item_tab.at[idx_i.at[pl.ds(o, ch)]],
                                  rows_i, s_gi).wait()
            pltpu.make_async_copy(pg_sh.at[idx_g.at[pl.ds(o, ch)]],
                                  rows_g, s_gg).wait()
            l0 = lax.rem(r0, _L)

            def row_body(i, rcarry):
                l = l0 + i
                for jj in range(_D // _LANES):
                    sl = pl.ds(jj * _LANES, _LANES)
                    plsc.addupdate(rows_i.at[i, sl],
                                   rows_g[i, sl] + pos_v[l, sl])
                return rcarry

            lax.fori_loop(0, ch, row_body, 0, unroll=8)
            pltpu.async_copy(rows_i, out.at[pl.ds(r0, ch)], s_sc)

        fetch(0, sets[0])

        def pair_body(g, carry):
            for b in range(2):
                c = 2 * g + b
                st, tt = sets[b], sets[1 - b]

                @pl.when(c >= 1)
                def _wait_scatter():
                    pltpu.make_async_copy(tt[0], out.at[pl.ds(base, ch)],
                                          tt[4]).wait()

                @pl.when(c + 1 < nch)
                def _prefetch():
                    fetch(c + 1, tt)

                process(c, st)
            return carry

        lax.fori_loop(0, nch // 2, pair_body, 0, unroll=False)
        pltpu.make_async_copy(sets[1][0], out.at[pl.ds(base, ch)],
                              sets[1][4]).wait()

    return sc_kernel


@jax.jit
def kernel(hist_item_id, hist_genre_id, item_table, genre_table, proj_W,
           proj_b, pos_table):
    item_idx = hist_item_id.reshape(-1).astype(jnp.int32)
    genre_idx = hist_genre_id.reshape(-1).astype(jnp.int32)
    table_lin = _tc_relayout(item_table.T).reshape(_IV, _D)
    pg, pb = _tc_prep(genre_table, proj_W, proj_b, pos_table)
    sc = _make_sc_kernel()
    out = sc(table_lin, pg, pb, item_idx, genre_idx)
    return out.reshape(_B, _L, _D)


# speedup vs baseline: 1.0489x; 1.0489x over previous
import functools
import jax
import jax.numpy as jnp
from jax import lax
from jax.experimental import pallas as pl
from jax.experimental.pallas import tpu as pltpu
from jax.experimental.pallas import tpu_sc as plsc

_B = 4096
_L = 200
_D = 64
_GV = 1000
_IV = 1000000
_ROWS = _B * _L
_LANES = 16
_RW = 8192


def _relayout_body(tin_ref, out_ref):
    x = tin_ref[...]
    y = jnp.swapaxes(x, 0, 1)
    y3 = y.reshape(_RW // 2, 2, _D)
    out_ref[...] = jnp.concatenate([y3[:, 0, :], y3[:, 1, :]], axis=1)


def _tc_relayout(item_table_t):
    return pl.pallas_call(
        _relayout_body,
        grid=(pl.cdiv(_IV, _RW),),
        in_specs=[pl.BlockSpec((_D, _RW), lambda i: (0, i))],
        out_specs=pl.BlockSpec((_RW // 2, 128), lambda i: (i, 0)),
        out_shape=jax.ShapeDtypeStruct((_IV // 2, 128), jnp.float32),
    )(item_table_t)


def _prep_body(genre_ref, w_ref, b_ref, pos_ref, pg_ref, pb_ref):
    pg_ref[...] = jnp.dot(genre_ref[...], w_ref[...],
                          preferred_element_type=jnp.float32)
    pb_ref[...] = pos_ref[...] + b_ref[...]


def _tc_prep(genre_table, proj_W, proj_b, pos_table):
    return pl.pallas_call(
        _prep_body,
        out_shape=(
            jax.ShapeDtypeStruct((_GV, _D), jnp.float32),
            jax.ShapeDtypeStruct((_L, _D), jnp.float32),
        ),
    )(genre_table, proj_W, proj_b.reshape(1, _D), pos_table)


def _make_sc_kernel():
    info = plsc.get_sparse_core_info()
    nc, ns = info.num_cores, info.num_subcores
    nw = nc * ns
    rpw = _ROWS // nw
    ch = 128
    nch = rpw // ch

    mesh = plsc.VectorSubcoreMesh(core_axis_name="c", subcore_axis_name="s")

    @functools.partial(
        pl.kernel,
        mesh=mesh,
        out_type=jax.ShapeDtypeStruct((_ROWS, _D), jnp.float32),
        compiler_params=pltpu.CompilerParams(use_tc_tiling_on_sc=False),
        scratch_types=(
            [pltpu.VMEM((ch,), jnp.int32) for _ in range(8)]
            + [pltpu.VMEM((ch, _D), jnp.float32) for _ in range(8)]
            + [
                pltpu.VMEM((2 * _L, _D), jnp.float32),
                pltpu.VMEM_SHARED((_GV, _D), jnp.float32),
            ]
            + [pltpu.SemaphoreType.DMA for _ in range(20)]
        ),
    )
    def sc_kernel(item_tab, pg_tab, pb_tab, item_idx, genre_idx, out, *scr):
        cid = lax.axis_index("c")
        sid = lax.axis_index("s")
        w = sid * nc + cid
        base = w * rpw
        pos_v, pg_sh = scr[16], scr[17]
        sets = tuple(
            (scr[2 * k], scr[2 * k + 1], scr[8 + 2 * k], scr[9 + 2 * k],
             scr[18 + 5 * k], scr[19 + 5 * k], scr[20 + 5 * k],
             scr[21 + 5 * k], scr[22 + 5 * k])
            for k in range(4)
        )

        @pl.when(sid == 0)
        def _fill_shared():
            pltpu.sync_copy(pg_tab, pg_sh)

        pltpu.sync_copy(pb_tab, pos_v.at[pl.ds(0, _L)])
        pltpu.sync_copy(pb_tab, pos_v.at[pl.ds(_L, _L)])
        plsc.subcore_barrier()

        def fetch_idx(c, st):
            r0 = base + c * ch
            pltpu.async_copy(item_idx.at[pl.ds(r0, ch)], st[0], st[4])
            pltpu.async_copy(genre_idx.at[pl.ds(r0, ch)], st[1], st[5])

        def wait_idx(c, st):
            r0 = base + c * ch
            pltpu.make_async_copy(item_idx.at[pl.ds(r0, ch)], st[0], st[4]).wait()
            pltpu.make_async_copy(genre_idx.at[pl.ds(r0, ch)], st[1], st[5]).wait()

        def fetch_rows(st):
            pltpu.async_copy(item_tab.at[st[0]], st[2], st[6])
            pltpu.async_copy(pg_sh.at[st[1]], st[3], st[7])

        def wait_scatter(st):
            pltpu.make_async_copy(st[2], out.at[pl.ds(base, ch)], st[8]).wait()

        def process(c, st):
            rows_i, rows_g = st[2], st[3]
            r0 = base + c * ch
            pltpu.make_async_copy(item_tab.at[st[0]], rows_i, st[6]).wait()
            pltpu.make_async_copy(pg_sh.at[st[1]], rows_g, st[7]).wait()
            l0 = lax.rem(r0, _L)

            def row_body(i, rcarry):
                l = l0 + i
                for jj in range(_D // _LANES):
                    sl = pl.ds(jj * _LANES, _LANES)
                    plsc.addupdate(rows_i.at[i, sl],
                                   rows_g[i, sl] + pos_v[l, sl])
                return rcarry

            lax.fori_loop(0, ch, row_body, 0, unroll=8)
            pltpu.async_copy(rows_i, out.at[pl.ds(r0, ch)], st[8])

        for k in range(3):
            fetch_idx(k, sets[k])
        for k in range(2):
            wait_idx(k, sets[k])
            fetch_rows(sets[k])

        def quad_body(g, carry):
            for b in range(4):
                c = 4 * g + b
                st = sets[b]
                st2 = sets[(b + 2) % 4]
                st3 = sets[(b + 3) % 4]

                @pl.when(c + 2 < nch)
                def _start_next_gather():
                    wait_idx(c + 2, st2)

                    @pl.when(c >= 2)
                    def _wait_prev_scatter():
                        wait_scatter(st2)

                    fetch_rows(st2)

                @pl.when(c + 3 < nch)
                def _start_next_idx():
                    fetch_idx(c + 3, st3)

                process(c, st)
            return carry

        lax.fori_loop(0, nch // 4, quad_body, 0, unroll=False)
        for k in range(4):
            wait_scatter(sets[k])

    return sc_kernel


@jax.jit
def kernel(hist_item_id, hist_genre_id, item_table, genre_table, proj_W,
           proj_b, pos_table):
    item_idx = hist_item_id.reshape(-1).astype(jnp.int32)
    genre_idx = hist_genre_id.reshape(-1).astype(jnp.int32)
    table_lin = _tc_relayout(item_table.T).reshape(_IV, _D)
    pg, pb = _tc_prep(genre_table, proj_W, proj_b, pos_table)
    sc = _make_sc_kernel()
    out = sc(table_lin, pg, pb, item_idx, genre_idx)
    return out.reshape(_B, _L, _D)
